# baseline (speedup 1.0000x reference)
import functools

import jax
import jax.numpy as jnp
from jax import lax
from jax.experimental import pallas as pl
from jax.experimental.pallas import tpu as pltpu
from jax.experimental.pallas import tpu_sc as plsc

D = 64
LANES = 16
NC, NS = 2, 16
NW = NC * NS
SCALE = 8.0

T = 200
S0 = 128
S1 = T - S0
TU = 4


def _build(n_b):
    assert n_b % NW == 0
    b_per_w = n_b // NW
    assert b_per_w % 2 == 0

    mesh = plsc.VectorSubcoreMesh(core_axis_name="c", subcore_axis_name="s")

    @functools.partial(
        pl.kernel,
        out_type=jax.ShapeDtypeStruct((n_b * T, 2 * D), jnp.float32),
        mesh=mesh,
        scratch_types=[
            pltpu.VMEM((2, T), jnp.int32),
            pltpu.VMEM((2, T, D), jnp.float32),
            pltpu.VMEM((2, T, 2 * D), jnp.float32),
            pltpu.SemaphoreType.DMA,
            pltpu.SemaphoreType.DMA,
            pltpu.SemaphoreType.DMA,
            pltpu.SemaphoreType.DMA,
            pltpu.SemaphoreType.DMA,
            pltpu.SemaphoreType.DMA,
        ],
        compiler_params=pltpu.CompilerParams(use_tc_tiling_on_sc=False),
    )
    def body(idx_hbm, table_hbm, out_hbm, idx_v, rows_v, pack_v,
             gsem0, gsem1, wsem0, wsem1, isem0, isem1):
        gsem = (gsem0, gsem1)
        wsem = (wsem0, wsem1)
        isem = (isem0, isem1)
        wid = lax.axis_index("s") * NC + lax.axis_index("c")
        b0 = wid * b_per_w

        def fire_gathers(pb, ci):
            pltpu.async_copy(
                table_hbm.at[idx_v.at[pb, pl.ds(0, S0)]],
                rows_v.at[pb, pl.ds(0, S0)],
                gsem[pb],
            )
            pltpu.async_copy(
                table_hbm.at[idx_v.at[pb, pl.ds(S0, S1)]],
                rows_v.at[pb, pl.ds(S0, S1)],
                gsem[pb],
            )

        def wait_gathers(pb):
            pltpu.make_async_copy(
                table_hbm.at[idx_v.at[pb, pl.ds(0, S0)]],
                rows_v.at[pb, pl.ds(0, S0)],
                gsem[pb],
            ).wait()
            pltpu.make_async_copy(
                table_hbm.at[idx_v.at[pb, pl.ds(S0, S1)]],
                rows_v.at[pb, pl.ds(S0, S1)],
                gsem[pb],
            ).wait()

        def start_idx_load(pb, ci):
            pltpu.async_copy(idx_hbm.at[b0 + ci], idx_v.at[pb], isem[pb])

        def wait_idx(pb):
            pltpu.make_async_copy(
                idx_hbm.at[b0], idx_v.at[pb], isem[pb]
            ).wait()

        def start_write(pb, ci):
            pltpu.async_copy(
                pack_v.at[pb],
                out_hbm.at[pl.ds((b0 + ci) * T, T)],
                wsem[pb],
            )

        def wait_write(pb):
            pltpu.make_async_copy(
                pack_v.at[pb], out_hbm.at[pl.ds(b0 * T, T)], wsem[pb]
            ).wait()

        def scale(pb):
            def scale_body(r, carry):
                base = r * TU
                for tt in range(TU):
                    t_idx = base + tt
                    for c in range(D // LANES):
                        src_sl = (pb, t_idx, pl.ds(c * LANES, LANES))
                        dst_sl = (pb, t_idx, pl.ds(c * LANES, LANES))
                        pack_v[dst_sl] = rows_v[src_sl] * SCALE
                return carry

            lax.fori_loop(0, T // TU, scale_body, 0)

        def step(pb, ci):
            nb = 1 - pb

            @pl.when(ci + 1 < b_per_w)
            def _():
                @pl.when(ci >= 1)
                def _():
                    wait_write(nb)

                wait_idx(nb)
                fire_gathers(nb, ci + 1)

            wait_gathers(pb)

            @pl.when(ci + 2 < b_per_w)
            def _():
                start_idx_load(pb, ci + 2)

            scale(pb)
            start_write(pb, ci)

        pltpu.sync_copy(idx_hbm.at[b0], idx_v.at[0])
        fire_gathers(0, 0)
        start_idx_load(1, 1)

        def loop_body(g, carry):
            step(0, 2 * g)
            step(1, 2 * g + 1)
            return carry

        lax.fori_loop(0, b_per_w // 2, loop_body, 0)

        wait_write(0)
        wait_write(1)

    return body


def kernel(x, table):
    n_b, t = x.shape
    assert t == T
    out2d = _build(n_b)(x.astype(jnp.int32), table)
    return out2d[:, :D].reshape(n_b, T, D)


# speedup vs baseline: 1.1550x; 1.1550x over previous
import functools

import jax
import jax.numpy as jnp
from jax import lax
from jax.experimental import pallas as pl
from jax.experimental.pallas import tpu as pltpu
from jax.experimental.pallas import tpu_sc as plsc

D = 64
LANES = 16
NC, NS = 2, 16
NW = NC * NS
SCALE = 8.0

T = 200
S0 = 128
S1 = T - S0
TU = 4


def _build(n_b):
    assert n_b % NW == 0
    b_per_w = n_b // NW
    assert b_per_w % 2 == 0

    mesh = plsc.VectorSubcoreMesh(core_axis_name="c", subcore_axis_name="s")

    @functools.partial(
        pl.kernel,
        out_type=jax.ShapeDtypeStruct((n_b * T, 2 * D), jnp.float32),
        mesh=mesh,
        scratch_types=[
            pltpu.VMEM((2, T), jnp.int32),
            pltpu.VMEM((2, T, D), jnp.float32),
            pltpu.VMEM((2, T, 2 * D), jnp.float32),
            pltpu.SemaphoreType.DMA,
            pltpu.SemaphoreType.DMA,
            pltpu.SemaphoreType.DMA,
            pltpu.SemaphoreType.DMA,
            pltpu.SemaphoreType.DMA,
            pltpu.SemaphoreType.DMA,
        ],
        compiler_params=pltpu.CompilerParams(use_tc_tiling_on_sc=False),
    )
    def body(idx_hbm, table_hbm, out_hbm, idx_v, rows_v, pack_v,
             gsem0, gsem1, wsem0, wsem1, isem0, isem1):
        gsem = (gsem0, gsem1)
        wsem = (wsem0, wsem1)
        isem = (isem0, isem1)
        wid = lax.axis_index("s") * NC + lax.axis_index("c")
        b0 = wid * b_per_w

        def fire_gathers(pb, ci):
            pltpu.async_copy(
                table_hbm.at[idx_v.at[pb, pl.ds(0, S0)]],
                rows_v.at[pb, pl.ds(0, S0)],
                gsem[pb],
            )
            pltpu.async_copy(
                table_hbm.at[idx_v.at[pb, pl.ds(S0, S1)]],
                rows_v.at[pb, pl.ds(S0, S1)],
                gsem[pb],
            )

        def wait_gathers(pb):
            pltpu.make_async_copy(
                table_hbm.at[idx_v.at[pb, pl.ds(0, S0)]],
                rows_v.at[pb, pl.ds(0, S0)],
                gsem[pb],
            ).wait()
            pltpu.make_async_copy(
                table_hbm.at[idx_v.at[pb, pl.ds(S0, S1)]],
                rows_v.at[pb, pl.ds(S0, S1)],
                gsem[pb],
            ).wait()

        def start_idx_load(pb, ci):
            pltpu.async_copy(idx_hbm.at[b0 + ci], idx_v.at[pb], isem[pb])

        def wait_idx(pb):
            pltpu.make_async_copy(
                idx_hbm.at[b0], idx_v.at[pb], isem[pb]
            ).wait()

        def start_write(pb, ci):
            pltpu.async_copy(
                pack_v.at[pb],
                out_hbm.at[pl.ds((b0 + ci) * T, T)],
                wsem[pb],
            )

        def wait_write(pb):
            pltpu.make_async_copy(
                pack_v.at[pb], out_hbm.at[pl.ds(b0 * T, T)], wsem[pb]
            ).wait()

        def scale(pb):
            def scale_body(r, carry):
                base = r * TU
                for tt in range(TU):
                    t_idx = base + tt
                    for c in range(D // LANES):
                        src_sl = (pb, t_idx, pl.ds(c * LANES, LANES))
                        dst_sl = (pb, t_idx, pl.ds(c * LANES, LANES))
                        pack_v[dst_sl] = rows_v[src_sl] * SCALE
                return carry

            lax.fori_loop(0, T // TU, scale_body, 0)

        def step(pb, ci):
            nb = 1 - pb

            @pl.when(ci + 1 < b_per_w)
            def _():
                wait_idx(nb)
                fire_gathers(nb, ci + 1)

            wait_gathers(pb)

            @pl.when(ci + 2 < b_per_w)
            def _():
                start_idx_load(pb, ci + 2)

            @pl.when(ci >= 2)
            def _():
                wait_write(pb)

            scale(pb)
            start_write(pb, ci)

        pltpu.sync_copy(idx_hbm.at[b0], idx_v.at[0])
        fire_gathers(0, 0)
        start_idx_load(1, 1)

        def loop_body(g, carry):
            step(0, 2 * g)
            step(1, 2 * g + 1)
            return carry

        lax.fori_loop(0, b_per_w // 2, loop_body, 0)

        wait_write(0)
        wait_write(1)

    return body


def kernel(x, table):
    n_b, t = x.shape
    assert t == T
    out2d = _build(n_b)(x.astype(jnp.int32), table)
    return out2d[:, :D].reshape(n_b, T, D)


# speedup vs baseline: 1.9213x; 1.6635x over previous
import functools

import jax
import jax.numpy as jnp
from jax import lax
from jax.experimental import pallas as pl
from jax.experimental.pallas import tpu as pltpu
from jax.experimental.pallas import tpu_sc as plsc

D = 64
LANES = 16
NC, NS = 2, 16
NW = NC * NS
SCALE = 8.0

T = 200
S0 = 128
S1 = T - S0
TU = 4


def _build(n_b):
    assert n_b % NW == 0
    b_per_w = n_b // NW
    assert b_per_w % 2 == 0

    mesh = plsc.VectorSubcoreMesh(core_axis_name="c", subcore_axis_name="s")

    @functools.partial(
        pl.kernel,
        out_type=jax.ShapeDtypeStruct((n_b * T, 2 * D), jnp.float32),
        mesh=mesh,
        scratch_types=[
            pltpu.VMEM((2, T), jnp.int32),
            pltpu.VMEM((2, T, D), jnp.float32),
            pltpu.SemaphoreType.DMA,
            pltpu.SemaphoreType.DMA,
            pltpu.SemaphoreType.DMA,
            pltpu.SemaphoreType.DMA,
            pltpu.SemaphoreType.DMA,
            pltpu.SemaphoreType.DMA,
        ],
        compiler_params=pltpu.CompilerParams(use_tc_tiling_on_sc=False),
    )
    def body(idx_hbm, table_hbm, out_hbm, idx_v, rows_v,
             gsem0, gsem1, wsem0, wsem1, isem0, isem1):
        gsem = (gsem0, gsem1)
        wsem = (wsem0, wsem1)
        isem = (isem0, isem1)
        wid = lax.axis_index("s") * NC + lax.axis_index("c")
        b0 = wid * b_per_w

        def fire_gathers(pb, ci):
            pltpu.async_copy(
                table_hbm.at[idx_v.at[pb, pl.ds(0, S0)]],
                rows_v.at[pb, pl.ds(0, S0)],
                gsem[pb],
            )
            pltpu.async_copy(
                table_hbm.at[idx_v.at[pb, pl.ds(S0, S1)]],
                rows_v.at[pb, pl.ds(S0, S1)],
                gsem[pb],
            )

        def wait_gathers(pb):
            pltpu.make_async_copy(
                table_hbm.at[idx_v.at[pb, pl.ds(0, S0)]],
                rows_v.at[pb, pl.ds(0, S0)],
                gsem[pb],
            ).wait()
            pltpu.make_async_copy(
                table_hbm.at[idx_v.at[pb, pl.ds(S0, S1)]],
                rows_v.at[pb, pl.ds(S0, S1)],
                gsem[pb],
            ).wait()

        def start_idx_load(pb, ci):
            pltpu.async_copy(idx_hbm.at[b0 + ci], idx_v.at[pb], isem[pb])

        def wait_idx(pb):
            pltpu.make_async_copy(
                idx_hbm.at[b0], idx_v.at[pb], isem[pb]
            ).wait()

        def start_write(pb, ci):
            pltpu.async_copy(
                rows_v.at[pb],
                out_hbm.at[pl.ds((b0 + ci) * T, T), pl.ds(0, D)],
                wsem[pb],
            )

        def wait_write(pb):
            pltpu.make_async_copy(
                rows_v.at[pb], out_hbm.at[pl.ds(b0 * T, T), pl.ds(0, D)], wsem[pb]
            ).wait()

        def scale(pb):
            def scale_body(r, carry):
                base = r * TU
                for tt in range(TU):
                    t_idx = base + tt
                    for c in range(D // LANES):
                        sl = (pb, t_idx, pl.ds(c * LANES, LANES))
                        rows_v[sl] = rows_v[sl] * SCALE
                return carry

            lax.fori_loop(0, T // TU, scale_body, 0)

        def step(pb, ci):
            nb = 1 - pb

            @pl.when(ci + 1 < b_per_w)
            def _():
                wait_idx(nb)
                fire_gathers(nb, ci + 1)

            wait_gathers(pb)

            @pl.when(ci + 2 < b_per_w)
            def _():
                start_idx_load(pb, ci + 2)

            @pl.when(ci >= 2)
            def _():
                wait_write(pb)

            scale(pb)
            start_write(pb, ci)

        pltpu.sync_copy(idx_hbm.at[b0], idx_v.at[0])
        fire_gathers(0, 0)
        start_idx_load(1, 1)

        def loop_body(g, carry):
            step(0, 2 * g)
            step(1, 2 * g + 1)
            return carry

        lax.fori_loop(0, b_per_w // 2, loop_body, 0)

        wait_write(0)
        wait_write(1)

    return body


def kernel(x, table):
    n_b, t = x.shape
    assert t == T
    out2d = _build(n_b)(x.astype(jnp.int32), table)
    return out2d[:, :D].reshape(n_b, T, D)


# speedup vs baseline: 1.9537x; 1.0169x over previous
import functools

import jax
import jax.numpy as jnp
from jax import lax
from jax.experimental import pallas as pl
from jax.experimental.pallas import tpu as pltpu
from jax.experimental.pallas import tpu_sc as plsc

D = 64
LANES = 16
NC, NS = 2, 16
NW = NC * NS
SCALE = 8.0

T = 200
S0 = 128
S1 = T - S0
TU = 4
NBUF = 4


def _build(n_b):
    assert n_b % NW == 0
    b_per_w = n_b // NW
    assert b_per_w % NBUF == 0 and b_per_w >= 2 * NBUF

    mesh = plsc.VectorSubcoreMesh(core_axis_name="c", subcore_axis_name="s")

    @functools.partial(
        pl.kernel,
        out_type=jax.ShapeDtypeStruct((n_b * T, 2 * D), jnp.float32),
        mesh=mesh,
        scratch_types=[
            pltpu.VMEM((NBUF, T), jnp.int32),
            pltpu.VMEM((NBUF, T, D), jnp.float32),
            [pltpu.SemaphoreType.DMA] * NBUF,
            [pltpu.SemaphoreType.DMA] * NBUF,
            [pltpu.SemaphoreType.DMA] * NBUF,
        ],
        compiler_params=pltpu.CompilerParams(use_tc_tiling_on_sc=False),
    )
    def body(idx_hbm, table_hbm, out_hbm, idx_v, rows_v, gsem, wsem, isem):
        wid = lax.axis_index("s") * NC + lax.axis_index("c")
        b0 = wid * b_per_w

        def fire_gathers(pb, ci):
            pltpu.async_copy(
                table_hbm.at[idx_v.at[pb, pl.ds(0, S0)]],
                rows_v.at[pb, pl.ds(0, S0)],
                gsem[pb],
            )
            pltpu.async_copy(
                table_hbm.at[idx_v.at[pb, pl.ds(S0, S1)]],
                rows_v.at[pb, pl.ds(S0, S1)],
                gsem[pb],
            )

        def wait_gathers(pb):
            pltpu.make_async_copy(
                table_hbm.at[idx_v.at[pb, pl.ds(0, S0)]],
                rows_v.at[pb, pl.ds(0, S0)],
                gsem[pb],
            ).wait()
            pltpu.make_async_copy(
                table_hbm.at[idx_v.at[pb, pl.ds(S0, S1)]],
                rows_v.at[pb, pl.ds(S0, S1)],
                gsem[pb],
            ).wait()

        def start_idx_load(pb, ci):
            pltpu.async_copy(idx_hbm.at[b0 + ci], idx_v.at[pb], isem[pb])

        def wait_idx(pb):
            pltpu.make_async_copy(
                idx_hbm.at[b0], idx_v.at[pb], isem[pb]
            ).wait()

        def start_write(pb, ci):
            pltpu.async_copy(
                rows_v.at[pb],
                out_hbm.at[pl.ds((b0 + ci) * T, T), pl.ds(0, D)],
                wsem[pb],
            )

        def wait_write(pb):
            pltpu.make_async_copy(
                rows_v.at[pb],
                out_hbm.at[pl.ds(b0 * T, T), pl.ds(0, D)],
                wsem[pb],
            ).wait()

        def scale(pb):
            def scale_body(r, carry):
                base = r * TU
                for tt in range(TU):
                    for c in range(D // LANES):
                        sl = (pb, base + tt, pl.ds(c * LANES, LANES))
                        rows_v[sl] = rows_v[sl] * SCALE
                return carry

            lax.fori_loop(0, T // TU, scale_body, 0)

        def step(pb, ci):
            gb = (pb + 2) % NBUF

            @pl.when(ci + 2 < b_per_w)
            def _():
                @pl.when(ci >= 2)
                def _():
                    wait_write(gb)

                wait_idx(gb)
                fire_gathers(gb, ci + 2)

            wait_gathers(pb)

            @pl.when(ci + NBUF < b_per_w)
            def _():
                start_idx_load(pb, ci + NBUF)

            scale(pb)
            start_write(pb, ci)

        pltpu.sync_copy(idx_hbm.at[b0], idx_v.at[0])
        fire_gathers(0, 0)
        start_idx_load(1, 1)
        start_idx_load(2, 2)
        start_idx_load(3, 3)
        wait_idx(1)
        fire_gathers(1, 1)

        def loop_body(g, carry):
            for pb in range(NBUF):
                step(pb, NBUF * g + pb)
            return carry

        lax.fori_loop(0, b_per_w // NBUF, loop_body, 0)

        for pb in range(NBUF):
            wait_write((b_per_w - NBUF + pb) % NBUF)

    return body


def kernel(x, table):
    n_b, t = x.shape
    assert t == T
    out2d = _build(n_b)(x.astype(jnp.int32), table)
    return out2d[:, :D].reshape(n_b, T, D)
